# baseline (speedup 1.0000x reference)
import functools

import jax
import jax.numpy as jnp
from jax import lax
from jax.experimental import pallas as pl
from jax.experimental.pallas import tpu as pltpu
from jax.experimental.pallas import tpu_sc as plsc

NC, NS = 2, 16
NW = NC * NS
LANES = 16

_SC_PARAMS = pltpu.CompilerParams(
    use_tc_tiling_on_sc=False, needs_layout_passes=False)


def _wid():
    return lax.axis_index("s") * NC + lax.axis_index("c")


@functools.lru_cache(maxsize=None)
def _make_dy_call(B, L, T, D):
    BPW = B // NW
    PER = BPW * L
    mesh = plsc.VectorSubcoreMesh(core_axis_name="c", subcore_axis_name="s")
    emb = jax.ShapeDtypeStruct((B, L, D), jnp.float32)

    @functools.partial(
        pl.kernel,
        out_type=[emb] * 3,
        mesh=mesh,
        compiler_params=_SC_PARAMS,
        scratch_types=[
            pltpu.VMEM((BPW, L), jnp.int32),
            pltpu.VMEM((BPW, LANES), jnp.int32),
            pltpu.VMEM((L, T, D), jnp.float32),
            pltpu.VMEM((L, T, D), jnp.float32),
            pltpu.VMEM((L, D), jnp.float32),
            pltpu.VMEM((L, D), jnp.float32),
            pltpu.VMEM((L, D), jnp.float32),
            pltpu.VMEM((L, D), jnp.float32),
            pltpu.VMEM((L, D), jnp.float32),
            pltpu.VMEM((L, D), jnp.float32),
            pltpu.SemaphoreType.DMA,
            pltpu.SemaphoreType.DMA,
            pltpu.SemaphoreType.DMA,
            pltpu.SemaphoreType.DMA,
        ],
    )
    def dy_call(road_hbm, time_hbm, tab3,
                o0, o1, o2,
                road_v, time_v, blk_a, blk_b,
                s0a, s0b, s1a, s1b, s2a, s2b,
                g_a, g_b, w_a, w_b):
        w = _wid()
        tb = w * BPW
        pltpu.sync_copy(road_hbm.at[pl.ds(tb, BPW)], road_v)
        pltpu.sync_copy(time_hbm.at[pl.ds(tb, BPW)], time_v)

        blk = (blk_a, blk_b)
        sb = ((s0a, s1a, s2a), (s0b, s1b, s2b))
        gsem = (g_a, g_b)
        wsem = (w_a, w_b)
        iota = lax.iota(jnp.int32, LANES)

        def start_gather(i, s):
            cp = pltpu.make_async_copy(
                tab3.at[road_v.at[i]], blk[s], gsem[s])
            cp.start()
            return cp

        pend_g = [start_gather(0, 0), None]
        pend_w = [[], []]
        for i in range(BPW):
            s = i & 1
            pend_g[s].wait()
            if i + 1 < BPW:
                pend_g[1 - s] = start_gather(i + 1, 1 - s)
            for cp in pend_w[s]:
                cp.wait()
            pend_w[s] = []

            t0 = time_v[i]
            t1 = jnp.where(t0 + 1 >= T, t0 + 1 - T, t0 + 1)
            t2 = jnp.where(t0 + 2 >= T, t0 + 2 - T, t0 + 2)
            bcur, (c0, c1, c2) = blk[s], sb[s]

            def row(l, carry):
                lv = jnp.full((LANES,), l, jnp.int32)
                rv = jnp.full((LANES,), L - 1, jnp.int32) - lv
                ov = jnp.where(lv - 1 < 0, lv - 1 + L, lv - 1)
                for k in range(D // LANES):
                    d = k * LANES + iota
                    sl = pl.ds(k * LANES, LANES)
                    c0[l, sl] = plsc.load_gather(bcur, [lv, t0, d])
                    c1[l, sl] = plsc.load_gather(bcur, [rv, t1, d])
                    c2[l, sl] = plsc.load_gather(bcur, [ov, t2, d])
                return carry

            lax.fori_loop(0, L, row, 0)

            handles = []
            for cbuf, oref in ((c0, o0), (c1, o1), (c2, o2)):
                cp = pltpu.make_async_copy(cbuf, oref.at[tb + i], wsem[s])
                cp.start()
                handles.append(cp)
            pend_w[s] = handles
        for s in (0, 1):
            for cp in pend_w[s]:
                cp.wait()

    return dy_call


@functools.lru_cache(maxsize=None)
def _make_static_call(B, L, D):
    BPW = B // NW
    PER = BPW * L
    mesh = plsc.VectorSubcoreMesh(core_axis_name="c", subcore_axis_name="s")
    emb = jax.ShapeDtypeStruct((B, L, D), jnp.float32)

    @functools.partial(
        pl.kernel,
        out_type=[emb] * 3,
        mesh=mesh,
        compiler_params=_SC_PARAMS,
        scratch_types=[
            pltpu.VMEM((BPW, L), jnp.int32),
            pltpu.VMEM((L, D), jnp.float32),
            pltpu.VMEM((L, D), jnp.float32),
            pltpu.VMEM((L, D), jnp.float32),
            pltpu.VMEM((L, D), jnp.float32),
            pltpu.SemaphoreType.DMA,
            pltpu.SemaphoreType.DMA,
            pltpu.SemaphoreType.DMA,
            pltpu.SemaphoreType.DMA,
        ],
    )
    def static_call(idx_hbm, tab,
                    o_id, o_rev, o_roll,
                    idx_v, buf_a, buf_b, rbuf_a, rbuf_b,
                    g_a, g_b, w_a, w_b):
        w = _wid()
        tb = w * BPW
        pltpu.sync_copy(idx_hbm.at[pl.ds(tb, BPW)], idx_v)

        buf = (buf_a, buf_b)
        rbuf = (rbuf_a, rbuf_b)
        gsem = (g_a, g_b)
        wsem = (w_a, w_b)
        iota = lax.iota(jnp.int32, LANES)

        def start_gather(i, s):
            cp = pltpu.make_async_copy(
                tab.at[idx_v.at[i]], buf[s], gsem[s])
            cp.start()
            return cp

        pend_g = [start_gather(0, 0), None]
        pend_w = [[], []]
        for i in range(BPW):
            s = i & 1
            pend_g[s].wait()
            for cp in pend_w[1 - s]:
                cp.wait()
            pend_w[1 - s] = []
            if i + 1 < BPW:
                pend_g[1 - s] = start_gather(i + 1, 1 - s)
            bcur, rcur = buf[s], rbuf[s]

            def row(l, carry):
                rv = jnp.full((LANES,), L - 1, jnp.int32) \
                    - jnp.full((LANES,), l, jnp.int32)
                for k in range(D // LANES):
                    d = k * LANES + iota
                    rcur[l, pl.ds(k * LANES, LANES)] = \
                        plsc.load_gather(bcur, [rv, d])
                return carry

            lax.fori_loop(0, L, row, 0)

            handles = []
            for src, dst in (
                    (bcur, o_id.at[tb + i]),
                    (rcur, o_rev.at[tb + i]),
                    (bcur.at[pl.ds(0, L - 1)],
                     o_roll.at[tb + i, pl.ds(1, L - 1)]),
                    (bcur.at[pl.ds(L - 1, 1)],
                     o_roll.at[tb + i, pl.ds(0, 1)]),
            ):
                cp = pltpu.make_async_copy(src, dst, wsem[s])
                cp.start()
                handles.append(cp)
            pend_w[s] = handles
        for s in (0, 1):
            for cp in pend_w[s]:
                cp.wait()

    return static_call


def kernel(road_idx, cell_idx, time_idx, dytraffic_embs, road_emb2, cell_embs):
    B, L = road_idx.shape
    R, T, D = dytraffic_embs.shape

    road2d = road_idx.astype(jnp.int32)
    cell2d = cell_idx.astype(jnp.int32)
    tvec = time_idx.astype(jnp.int32)

    tsplat = jnp.broadcast_to(tvec[:, None], (B, LANES))
    dy0, dy1, dy2 = _make_dy_call(B, L, T, D)(road2d, tsplat, dytraffic_embs)
    r0, ra, rb = _make_static_call(B, L, D)(road2d, road_emb2)
    c0, c1, c2 = _make_static_call(B, L, D)(cell2d, cell_embs)

    lens = jnp.full((B,), L, dtype=jnp.int32)
    t1 = (tvec + 1) % T
    t2 = (tvec + 2) % T
    times = jnp.broadcast_to(tvec[:, None], (B, L))
    times1 = jnp.broadcast_to(t1[:, None], (B, L))
    times2 = jnp.broadcast_to(t2[:, None], (B, L))

    return (dy1, lens, dy2, lens, dy0, lens,
            ra, lens, rb, lens, r0, lens,
            c1, lens, c2, lens, c0, lens,
            times1, times2, times)
